# baseline (speedup 1.0000x reference)
import jax
import jax.numpy as jnp
from jax.experimental import pallas as pl
from jax.experimental.pallas import tpu as pltpu

B, N, F_ATOM, F_PHYS = 64, 512, 38, 15
R_OUT, C_OUT = 20, 128

G = 4

_SCORE_SCALE = float((1.4426950408889634 * F_ATOM ** -0.5) ** 0.5)


def _body_kernel(x_ref, wr_ref, br_ref, wc_ref, bc_ref, g_ref, ph_ref):
    for i in range(G):
        blk = x_ref[i]
        x = blk[:, :F_ATOM]
        ph_ref[i] = blk[0:1, F_ATOM:]
        xs = (x * jnp.float32(_SCORE_SCALE)).astype(jnp.bfloat16)
        s = jax.lax.dot_general(xs, xs, (((1,), (1,)), ((), ())),
                                preferred_element_type=jnp.float32)
        p = jnp.exp2(s)
        denom = jnp.sum(p, axis=1, keepdims=True)
        xr = jnp.dot(x, wr_ref[...],
                     preferred_element_type=jnp.float32)
        u = jnp.dot(p.astype(jnp.bfloat16), xr.astype(jnp.bfloat16),
                    preferred_element_type=jnp.float32)
        hr = jnp.maximum(u / denom + br_ref[...], 0.0)
        h = jnp.maximum(
            jnp.dot(hr, wc_ref[...], preferred_element_type=jnp.float32)
            + bc_ref[...], 0.0)
        g_ref[i] = jnp.sum(h, axis=0, keepdims=True) * jnp.float32(1.0 / N)


def _head_kernel(g_ref, phys_ref, w1_ref, b1_ref, w5_ref, b5_ref,
                 w6t_ref, b6_ref, w7h_ref, w7pt_ref, b7_ref, out_ref):
    g = g_ref[:, 0, :]
    phys = phys_ref[:, 0, :]
    z = jnp.maximum(
        jnp.dot(g, w1_ref[...], preferred_element_type=jnp.float32)
        + b1_ref[...], 0.0)
    z = jnp.maximum(
        jnp.dot(z, w5_ref[...], preferred_element_type=jnp.float32)
        + b5_ref[...], 0.0)
    mv = jnp.sum(z * w6t_ref[...], axis=1, keepdims=True) + b6_ref[...]
    o = (mv * w7h_ref[...]
         + jnp.sum(phys * w7pt_ref[...], axis=1, keepdims=True)
         + b7_ref[...])
    out_ref[...] = jnp.concatenate([o, phys], axis=1)


def kernel(inputs, Wr, br, Wc, bc, W1, b1, W5, b5, W6, b6, W7, b7):
    def rep(a):
        return pl.BlockSpec(a.shape, lambda b: (0,) * a.ndim)

    br2 = br.reshape(1, R_OUT)
    bc2 = bc.reshape(1, C_OUT)

    g, phys = pl.pallas_call(
        _body_kernel,
        grid=(B // G,),
        in_specs=[
            pl.BlockSpec((G, N, F_ATOM + F_PHYS), lambda b: (b, 0, 0)),
            rep(Wr), rep(br2), rep(Wc), rep(bc2),
        ],
        out_specs=[
            pl.BlockSpec((G, 1, C_OUT), lambda b: (b, 0, 0)),
            pl.BlockSpec((G, 1, F_PHYS), lambda b: (b, 0, 0)),
        ],
        out_shape=[
            jax.ShapeDtypeStruct((B, 1, C_OUT), jnp.float32),
            jax.ShapeDtypeStruct((B, 1, F_PHYS), jnp.float32),
        ],
        compiler_params=pltpu.CompilerParams(
            dimension_semantics=("parallel",),
        ),
    )(inputs, Wr, br2, Wc, bc2)

    b12 = b1.reshape(1, 32)
    b52 = b5.reshape(1, 16)
    b62 = b6.reshape(1, 1)
    b72 = b7.reshape(1, 1)
    w6t = W6.reshape(1, 16)
    w7h = W7[0].reshape(1, 1)
    w7pt = W7[1:, 0].reshape(1, F_PHYS)
    head_ws = (W1, b12, W5, b52, w6t, b62, w7h, w7pt, b72)

    def rep0(a):
        return pl.BlockSpec(a.shape, lambda: (0,) * a.ndim)

    return pl.pallas_call(
        _head_kernel,
        in_specs=[rep0(g), rep0(phys)] + [rep0(w) for w in head_ws],
        out_specs=pl.BlockSpec((B, 16), lambda: (0, 0)),
        out_shape=jax.ShapeDtypeStruct((B, 16), jnp.float32),
    )(g, phys, *head_ws)


# speedup vs baseline: 1.0531x; 1.0531x over previous
import jax
import jax.numpy as jnp
from jax.experimental import pallas as pl
from jax.experimental.pallas import tpu as pltpu

B, N, F_ATOM, F_PHYS = 64, 512, 38, 15
R_OUT, C_OUT = 20, 128

G = 8

_SCORE_SCALE = float((1.4426950408889634 * F_ATOM ** -0.5) ** 0.5)


def _body_kernel(x_ref, wr_ref, br_ref, wc_ref, bc_ref, g_ref, ph_ref):
    for i in range(G):
        blk = x_ref[i]
        x = blk[:, :F_ATOM]
        ph_ref[i] = blk[0:1, F_ATOM:]
        xs = (x * jnp.float32(_SCORE_SCALE)).astype(jnp.bfloat16)
        s = jax.lax.dot_general(xs, xs, (((1,), (1,)), ((), ())),
                                preferred_element_type=jnp.float32)
        p = jnp.exp2(s)
        denom = jnp.sum(p, axis=1, keepdims=True)
        xr = jnp.dot(x, wr_ref[...],
                     preferred_element_type=jnp.float32)
        u = jnp.dot(p.astype(jnp.bfloat16), xr.astype(jnp.bfloat16),
                    preferred_element_type=jnp.float32)
        hr = jnp.maximum(u / denom + br_ref[...], 0.0)
        h = jnp.maximum(
            jnp.dot(hr, wc_ref[...], preferred_element_type=jnp.float32)
            + bc_ref[...], 0.0)
        g_ref[i] = jnp.sum(h, axis=0, keepdims=True) * jnp.float32(1.0 / N)


def _head_kernel(g_ref, phys_ref, w1_ref, b1_ref, w5_ref, b5_ref,
                 w6t_ref, b6_ref, w7h_ref, w7pt_ref, b7_ref, out_ref):
    g = g_ref[:, 0, :]
    phys = phys_ref[:, 0, :]
    z = jnp.maximum(
        jnp.dot(g, w1_ref[...], preferred_element_type=jnp.float32)
        + b1_ref[...], 0.0)
    z = jnp.maximum(
        jnp.dot(z, w5_ref[...], preferred_element_type=jnp.float32)
        + b5_ref[...], 0.0)
    mv = jnp.sum(z * w6t_ref[...], axis=1, keepdims=True) + b6_ref[...]
    o = (mv * w7h_ref[...]
         + jnp.sum(phys * w7pt_ref[...], axis=1, keepdims=True)
         + b7_ref[...])
    out_ref[...] = jnp.concatenate([o, phys], axis=1)


def kernel(inputs, Wr, br, Wc, bc, W1, b1, W5, b5, W6, b6, W7, b7):
    def rep(a):
        return pl.BlockSpec(a.shape, lambda b: (0,) * a.ndim)

    br2 = br.reshape(1, R_OUT)
    bc2 = bc.reshape(1, C_OUT)

    g, phys = pl.pallas_call(
        _body_kernel,
        grid=(B // G,),
        in_specs=[
            pl.BlockSpec((G, N, F_ATOM + F_PHYS), lambda b: (b, 0, 0)),
            rep(Wr), rep(br2), rep(Wc), rep(bc2),
        ],
        out_specs=[
            pl.BlockSpec((G, 1, C_OUT), lambda b: (b, 0, 0)),
            pl.BlockSpec((G, 1, F_PHYS), lambda b: (b, 0, 0)),
        ],
        out_shape=[
            jax.ShapeDtypeStruct((B, 1, C_OUT), jnp.float32),
            jax.ShapeDtypeStruct((B, 1, F_PHYS), jnp.float32),
        ],
        compiler_params=pltpu.CompilerParams(
            dimension_semantics=("parallel",),
        ),
    )(inputs, Wr, br2, Wc, bc2)

    b12 = b1.reshape(1, 32)
    b52 = b5.reshape(1, 16)
    b62 = b6.reshape(1, 1)
    b72 = b7.reshape(1, 1)
    w6t = W6.reshape(1, 16)
    w7h = W7[0].reshape(1, 1)
    w7pt = W7[1:, 0].reshape(1, F_PHYS)
    head_ws = (W1, b12, W5, b52, w6t, b62, w7h, w7pt, b72)

    def rep0(a):
        return pl.BlockSpec(a.shape, lambda: (0,) * a.ndim)

    return pl.pallas_call(
        _head_kernel,
        in_specs=[rep0(g), rep0(phys)] + [rep0(w) for w in head_ws],
        out_specs=pl.BlockSpec((B, 16), lambda: (0, 0)),
        out_shape=jax.ShapeDtypeStruct((B, 16), jnp.float32),
    )(g, phys, *head_ws)


# speedup vs baseline: 1.0676x; 1.0138x over previous
import jax
import jax.numpy as jnp
from jax.experimental import pallas as pl
from jax.experimental.pallas import tpu as pltpu

B, N, F_ATOM, F_PHYS = 64, 512, 38, 15
R_OUT, C_OUT = 20, 128

G = 16

_SCORE_SCALE = float((1.4426950408889634 * F_ATOM ** -0.5) ** 0.5)


def _body_kernel(x_ref, wr_ref, br_ref, wc_ref, bc_ref, g_ref, ph_ref):
    for i in range(G):
        blk = x_ref[i]
        x = blk[:, :F_ATOM]
        ph_ref[i] = blk[0:1, F_ATOM:]
        xs = (x * jnp.float32(_SCORE_SCALE)).astype(jnp.bfloat16)
        s = jax.lax.dot_general(xs, xs, (((1,), (1,)), ((), ())),
                                preferred_element_type=jnp.float32)
        p = jnp.exp2(s)
        denom = jnp.sum(p, axis=1, keepdims=True)
        xr = jnp.dot(x, wr_ref[...],
                     preferred_element_type=jnp.float32)
        u = jnp.dot(p.astype(jnp.bfloat16), xr.astype(jnp.bfloat16),
                    preferred_element_type=jnp.float32)
        hr = jnp.maximum(u / denom + br_ref[...], 0.0)
        h = jnp.maximum(
            jnp.dot(hr, wc_ref[...], preferred_element_type=jnp.float32)
            + bc_ref[...], 0.0)
        g_ref[i] = jnp.sum(h, axis=0, keepdims=True) * jnp.float32(1.0 / N)


def _head_kernel(g_ref, phys_ref, w1_ref, b1_ref, w5_ref, b5_ref,
                 w6t_ref, b6_ref, w7h_ref, w7pt_ref, b7_ref, out_ref):
    g = g_ref[:, 0, :]
    phys = phys_ref[:, 0, :]
    z = jnp.maximum(
        jnp.dot(g, w1_ref[...], preferred_element_type=jnp.float32)
        + b1_ref[...], 0.0)
    z = jnp.maximum(
        jnp.dot(z, w5_ref[...], preferred_element_type=jnp.float32)
        + b5_ref[...], 0.0)
    mv = jnp.sum(z * w6t_ref[...], axis=1, keepdims=True) + b6_ref[...]
    o = (mv * w7h_ref[...]
         + jnp.sum(phys * w7pt_ref[...], axis=1, keepdims=True)
         + b7_ref[...])
    out_ref[...] = jnp.concatenate([o, phys], axis=1)


def kernel(inputs, Wr, br, Wc, bc, W1, b1, W5, b5, W6, b6, W7, b7):
    def rep(a):
        return pl.BlockSpec(a.shape, lambda b: (0,) * a.ndim)

    br2 = br.reshape(1, R_OUT)
    bc2 = bc.reshape(1, C_OUT)

    g, phys = pl.pallas_call(
        _body_kernel,
        grid=(B // G,),
        in_specs=[
            pl.BlockSpec((G, N, F_ATOM + F_PHYS), lambda b: (b, 0, 0)),
            rep(Wr), rep(br2), rep(Wc), rep(bc2),
        ],
        out_specs=[
            pl.BlockSpec((G, 1, C_OUT), lambda b: (b, 0, 0)),
            pl.BlockSpec((G, 1, F_PHYS), lambda b: (b, 0, 0)),
        ],
        out_shape=[
            jax.ShapeDtypeStruct((B, 1, C_OUT), jnp.float32),
            jax.ShapeDtypeStruct((B, 1, F_PHYS), jnp.float32),
        ],
        compiler_params=pltpu.CompilerParams(
            dimension_semantics=("parallel",),
        ),
    )(inputs, Wr, br2, Wc, bc2)

    b12 = b1.reshape(1, 32)
    b52 = b5.reshape(1, 16)
    b62 = b6.reshape(1, 1)
    b72 = b7.reshape(1, 1)
    w6t = W6.reshape(1, 16)
    w7h = W7[0].reshape(1, 1)
    w7pt = W7[1:, 0].reshape(1, F_PHYS)
    head_ws = (W1, b12, W5, b52, w6t, b62, w7h, w7pt, b72)

    def rep0(a):
        return pl.BlockSpec(a.shape, lambda: (0,) * a.ndim)

    return pl.pallas_call(
        _head_kernel,
        in_specs=[rep0(g), rep0(phys)] + [rep0(w) for w in head_ws],
        out_specs=pl.BlockSpec((B, 16), lambda: (0, 0)),
        out_shape=jax.ShapeDtypeStruct((B, 16), jnp.float32),
    )(g, phys, *head_ws)
